# Initial kernel scaffold; baseline (speedup 1.0000x reference)
import jax
import jax.numpy as jnp
from jax.experimental import pallas as pl


def kernel(sparse_features, dense_features, sparse_w, sparse_emb, dw_W, dw_b, de_W, de_b, bias):
    raise NotImplementedError("write your pallas kernel here")


# speedup vs baseline: 1.1313x; 1.1313x over previous
import functools

import jax
import jax.numpy as jnp
from jax import lax
from jax.experimental import pallas as pl
from jax.experimental.pallas import tpu as pltpu
from jax.experimental.pallas import tpu_sc as plsc

NFIELD = 26
VOCAB = 100000
EMB = 16
BATCH = 4096
NCORE = 2
NSUB = 16
NWORK = NCORE * NSUB
BPW = BATCH // NWORK
IDX_CHUNK = 128
NCHUNK = BPW * NFIELD // IDX_CHUNK
GROUPS = BPW // 16


def _fm_body(gidx_hbm, widx_hbm, emb_hbm, w_hbm, dproj_hbm, dlin_hbm, out_hbm,
             idx_v, widx_v, rows_v, w_v, dproj_v, dlin_v, out_v, esem, wsem):
    wid = lax.axis_index("s") * NCORE + lax.axis_index("c")
    base = wid * BPW

    pltpu.sync_copy(gidx_hbm.at[wid], idx_v)
    pltpu.sync_copy(widx_hbm.at[wid], widx_v)
    pltpu.sync_copy(dproj_hbm.at[pl.ds(base, BPW)], dproj_v)
    pltpu.sync_copy(dlin_hbm.at[pl.ds(base, BPW)], dlin_v)

    copies = []
    for c in range(NCHUNK):
        copies.append(pltpu.async_copy(
            emb_hbm.at[idx_v.at[c]], rows_v.at[pl.ds(c * IDX_CHUNK, IDX_CHUNK)],
            esem))
        copies.append(pltpu.async_copy(
            w_hbm.at[widx_v.at[c]], w_v.at[pl.ds(c * IDX_CHUNK, IDX_CHUNK)],
            wsem))
    for cp in copies:
        cp.wait()

    lane = lax.iota(jnp.int32, 16)

    def group_body(g, carry):
        g16 = pl.multiple_of(g * 16, 16)
        fm = jnp.zeros((16,), jnp.float32)
        for i in range(16):
            row0 = (g16 + i) * NFIELD
            acc_s = dproj_v[g16 + i, :]
            acc_q = acc_s * acc_s
            for f in range(NFIELD):
                r = rows_v[row0 + f, :]
                acc_s = acc_s + r
                acc_q = acc_q + r * r
            fm = jnp.where(lane == i, jnp.sum(acc_s * acc_s - acc_q), fm)
        lin = dlin_v[pl.ds(g16, 16)]
        for f in range(NFIELD):
            lin = lin + w_v[pl.ds(f * BPW + g16, 16)]
        logit = lin + 0.5 * fm
        out_v[pl.ds(g16, 16)] = 1.0 / (1.0 + jnp.exp(-logit))
        return carry

    lax.fori_loop(0, GROUPS, group_body, 0)
    pltpu.sync_copy(out_v, out_hbm.at[pl.ds(base, BPW)])


@functools.partial(
    pl.kernel,
    out_type=jax.ShapeDtypeStruct((BATCH,), jnp.float32),
    mesh=plsc.VectorSubcoreMesh(core_axis_name="c", subcore_axis_name="s"),
    compiler_params=pltpu.CompilerParams(
        needs_layout_passes=False, use_tc_tiling_on_sc=False),
    scratch_types=[
        pltpu.VMEM((NCHUNK, IDX_CHUNK), jnp.int32),
        pltpu.VMEM((NCHUNK, IDX_CHUNK), jnp.int32),
        pltpu.VMEM((BPW * NFIELD, EMB), jnp.float32),
        pltpu.VMEM((NCHUNK * IDX_CHUNK,), jnp.float32),
        pltpu.VMEM((BPW, EMB), jnp.float32),
        pltpu.VMEM((BPW,), jnp.float32),
        pltpu.VMEM((BPW,), jnp.float32),
        pltpu.SemaphoreType.DMA,
        pltpu.SemaphoreType.DMA,
    ],
)
def _fm_call(gidx_hbm, widx_hbm, emb_hbm, w_hbm, dproj_hbm, dlin_hbm, out_hbm,
             idx_v, widx_v, rows_v, w_v, dproj_v, dlin_v, out_v, esem, wsem):
    _fm_body(gidx_hbm, widx_hbm, emb_hbm, w_hbm, dproj_hbm, dlin_hbm, out_hbm,
             idx_v, widx_v, rows_v, w_v, dproj_v, dlin_v, out_v, esem, wsem)


def kernel(sparse_features, dense_features, sparse_w, sparse_emb,
           dw_W, dw_b, de_W, de_b, bias):
    offs = (jnp.arange(NFIELD, dtype=jnp.int32) * VOCAB)[None, :]
    gflat = sparse_features.astype(jnp.int32) + offs
    gidx = gflat.reshape(NWORK, NCHUNK, IDX_CHUNK)
    widx = gflat.reshape(NWORK, BPW, NFIELD).transpose(0, 2, 1)
    emb_flat = sparse_emb.reshape(NFIELD * VOCAB, EMB)
    w_flat = sparse_w.reshape(NFIELD * VOCAB)
    dproj = dense_features @ de_W + de_b
    dlin = (dense_features @ dw_W)[:, 0] + dw_b[0] + bias[0]
    return _fm_call(gidx, widx, emb_flat, w_flat, dproj, dlin)


# speedup vs baseline: 1.1341x; 1.0024x over previous
import functools

import jax
import jax.numpy as jnp
from jax import lax
from jax.experimental import pallas as pl
from jax.experimental.pallas import tpu as pltpu
from jax.experimental.pallas import tpu_sc as plsc

NFIELD = 26
VOCAB = 100000
EMB = 16
BATCH = 4096
NCORE = 2
NSUB = 16
NWORK = NCORE * NSUB
BPW = BATCH // NWORK
GROUPS = BPW // 16


def _fm_body(vidx_hbm, emb_hbm, w_hbm, dproj_hbm, dlin_hbm, out_hbm,
             idx_v, rows_v, w_v, dproj_v, dlin_v, out_v, esem, wsem):
    wid = lax.axis_index("s") * NCORE + lax.axis_index("c")
    base = wid * BPW

    pltpu.sync_copy(vidx_hbm.at[wid], idx_v)
    pltpu.sync_copy(dproj_hbm.at[pl.ds(base, BPW)], dproj_v)
    pltpu.sync_copy(dlin_hbm.at[pl.ds(base, BPW)], dlin_v)

    copies = []
    for f in range(NFIELD):
        copies.append(pltpu.async_copy(
            emb_hbm.at[f].at[idx_v.at[f]], rows_v.at[pl.ds(f * BPW, BPW)],
            esem))
        copies.append(pltpu.async_copy(
            w_hbm.at[f].at[idx_v.at[f]], w_v.at[pl.ds(f * BPW, BPW)],
            wsem))
    for cp in copies:
        cp.wait()

    lane = lax.iota(jnp.int32, 16)

    def group_body(g, carry):
        g16 = pl.multiple_of(g * 16, 16)
        fm = jnp.zeros((16,), jnp.float32)
        for i in range(16):
            acc_s = dproj_v[g16 + i, :]
            acc_q = acc_s * acc_s
            for f in range(NFIELD):
                r = rows_v[f * BPW + g16 + i, :]
                acc_s = acc_s + r
                acc_q = acc_q + r * r
            fm = jnp.where(lane == i, jnp.sum(acc_s * acc_s - acc_q), fm)
        lin = dlin_v[pl.ds(g16, 16)]
        for f in range(NFIELD):
            lin = lin + w_v[pl.ds(f * BPW + g16, 16)]
        logit = lin + 0.5 * fm
        out_v[pl.ds(g16, 16)] = 1.0 / (1.0 + jnp.exp(-logit))
        return carry

    lax.fori_loop(0, GROUPS, group_body, 0)
    pltpu.sync_copy(out_v, out_hbm.at[pl.ds(base, BPW)])


@functools.partial(
    pl.kernel,
    out_type=jax.ShapeDtypeStruct((BATCH,), jnp.float32),
    mesh=plsc.VectorSubcoreMesh(core_axis_name="c", subcore_axis_name="s"),
    compiler_params=pltpu.CompilerParams(
        needs_layout_passes=False, use_tc_tiling_on_sc=False),
    scratch_types=[
        pltpu.VMEM((NFIELD, BPW), jnp.int32),
        pltpu.VMEM((NFIELD * BPW, EMB), jnp.float32),
        pltpu.VMEM((NFIELD * BPW,), jnp.float32),
        pltpu.VMEM((BPW, EMB), jnp.float32),
        pltpu.VMEM((BPW,), jnp.float32),
        pltpu.VMEM((BPW,), jnp.float32),
        pltpu.SemaphoreType.DMA,
        pltpu.SemaphoreType.DMA,
    ],
)
def _fm_call(vidx_hbm, emb_hbm, w_hbm, dproj_hbm, dlin_hbm, out_hbm,
             idx_v, rows_v, w_v, dproj_v, dlin_v, out_v, esem, wsem):
    _fm_body(vidx_hbm, emb_hbm, w_hbm, dproj_hbm, dlin_hbm, out_hbm,
             idx_v, rows_v, w_v, dproj_v, dlin_v, out_v, esem, wsem)


def kernel(sparse_features, dense_features, sparse_w, sparse_emb,
           dw_W, dw_b, de_W, de_b, bias):
    vidx = sparse_features.astype(jnp.int32).reshape(
        NWORK, BPW, NFIELD).transpose(0, 2, 1)
    w2 = sparse_w.reshape(NFIELD, VOCAB)
    dproj = dense_features @ de_W + de_b
    dlin = (dense_features @ dw_W)[:, 0] + dw_b[0] + bias[0]
    return _fm_call(vidx, sparse_emb, w2, dproj, dlin)
